# baseline (speedup 1.0000x reference)
import functools

import jax
import jax.numpy as jnp
from jax import lax
from jax.experimental import pallas as pl
from jax.experimental.pallas import tpu as pltpu
from jax.experimental.pallas import tpu_sc as plsc

E = 16
K = 2
H = 1024
F = 512
SCALE = 2.5
T = 4096
NSLOT = T * K
BT = 256
NBLK = 48
PAD = NBLK * BT

NC = 2
NS = 16
NW = NC * NS
CHUNK = 32
TOK_PER_W = T // NW
DCH = TOK_PER_W // CHUNK
SLOT_PER_W = NSLOT // NW
CCH = SLOT_PER_W // CHUNK


def _sigmoid(x):
    return 1.0 / (1.0 + jnp.exp(-x))


def _router_body(x_ref, wr_ref, idx_ref, w_ref):
    logits = lax.dot_general(x_ref[...], wr_ref[...],
                             (((1,), (1,)), ((), ())),
                             preferred_element_type=jnp.float32)
    scores = _sigmoid(logits)
    bt = scores.shape[0]
    iota = lax.broadcasted_iota(jnp.int32, (bt, E), 1)
    m1 = jnp.max(scores, axis=1, keepdims=True)
    i1 = jnp.min(jnp.where(scores == m1, iota, E), axis=1, keepdims=True)
    scores2 = jnp.where(iota == i1, -1.0, scores)
    m2 = jnp.max(scores2, axis=1, keepdims=True)
    i2 = jnp.min(jnp.where(scores2 == m2, iota, E), axis=1, keepdims=True)
    denom = m1 + m2 + 1e-20
    idx_ref[...] = jnp.concatenate([i1, i2], axis=1)
    w_ref[...] = jnp.concatenate([m1 / denom, m2 / denom], axis=1) * SCALE


def _router(x, wr):
    bt = 512
    return pl.pallas_call(
        _router_body,
        grid=(T // bt,),
        in_specs=[
            pl.BlockSpec((bt, H), lambda i: (i, 0)),
            pl.BlockSpec((E, H), lambda i: (0, 0)),
        ],
        out_specs=[
            pl.BlockSpec((bt, K), lambda i: (i, 0)),
            pl.BlockSpec((bt, K), lambda i: (i, 0)),
        ],
        out_shape=[
            jax.ShapeDtypeStruct((T, K), jnp.int32),
            jax.ShapeDtypeStruct((T, K), jnp.float32),
        ],
    )(x, wr)


_MR = 64
_MC = 128


def _metadata_body(e2d_ref, pos_ref, bex_ref):
    e2d = e2d_ref[...]
    r = lax.broadcasted_iota(jnp.int32, (_MC, _MC), 0)
    c = lax.broadcasted_iota(jnp.int32, (_MC, _MC), 1)
    upper_incl = (r <= c).astype(jnp.float32)
    r2 = lax.broadcasted_iota(jnp.int32, (_MR, _MR), 0)
    c2 = lax.broadcasted_iota(jnp.int32, (_MR, _MR), 1)
    lower_strict = (c2 < r2).astype(jnp.float32)

    counts = []
    for e in range(E):
        counts.append(jnp.sum((e2d == e).astype(jnp.float32)))
    pb = []
    acc = jnp.int32(0)
    blk_starts = []
    for e in range(E):
        blk_starts.append(acc)
        pb.append(acc * BT)
        nb = (counts[e].astype(jnp.int32) + (BT - 1)) // BT
        acc = acc + nb
    total_blocks = acc

    pos = jnp.zeros((_MR, _MC), jnp.float32)
    for e in range(E):
        m = (e2d == e).astype(jnp.float32)
        incl = lax.dot_general(m, upper_incl, (((1,), (0,)), ((), ())),
                               preferred_element_type=jnp.float32)
        tot = jnp.sum(m, axis=1, keepdims=True)
        ro = lax.dot_general(lower_strict, tot, (((1,), (0,)), ((), ())),
                             preferred_element_type=jnp.float32)
        rank = ro + incl - 1.0
        pos = pos + m * (pb[e].astype(jnp.float32) + rank)
    pos_ref[...] = pos.astype(jnp.int32)

    b_iota = lax.broadcasted_iota(jnp.int32, (1, NBLK), 1)
    bex = jnp.zeros((1, NBLK), jnp.int32)
    last_e = jnp.int32(-1)
    for e in range(E):
        bex = bex + (b_iota >= blk_starts[e]).astype(jnp.int32)
        last_e = last_e + (blk_starts[e] < total_blocks).astype(jnp.int32)
    bex = bex - 1
    bex_ref[0:1, :] = jnp.where(b_iota >= total_blocks, last_e, bex)
    bex_ref[1:2, :] = jnp.full((1, NBLK), 1, jnp.int32) * total_blocks


def _metadata(e2d):
    return pl.pallas_call(
        _metadata_body,
        in_specs=[pl.BlockSpec((_MR, _MC), lambda: (0, 0))],
        out_specs=[
            pl.BlockSpec((_MR, _MC), lambda: (0, 0)),
            pl.BlockSpec((2, NBLK), lambda: (0, 0)),
        ],
        out_shape=[
            jax.ShapeDtypeStruct((_MR, _MC), jnp.int32),
            jax.ShapeDtypeStruct((2, NBLK), jnp.int32),
        ],
    )(e2d)


def _sc_mesh():
    return plsc.VectorSubcoreMesh(core_axis_name="c", subcore_axis_name="s",
                                  num_cores=NC, num_subcores=NS)


def _sc_dispatch(x, pe, po):
    @functools.partial(
        pl.kernel,
        out_type=jax.ShapeDtypeStruct((PAD, H), jnp.float32),
        mesh=_sc_mesh(),
        scratch_types=[
            pltpu.VMEM((DCH, CHUNK), jnp.int32),
            pltpu.VMEM((DCH, CHUNK), jnp.int32),
            pltpu.VMEM((CHUNK, H), jnp.float32),
            pltpu.VMEM((CHUNK, H), jnp.float32),
            pltpu.SemaphoreType.DMA,
            pltpu.SemaphoreType.DMA,
        ],
    )
    def body(x_hbm, pe_hbm, po_hbm, xs_hbm, pe_v, po_v, rows0, rows1, gsem, ssem):
        wid = lax.axis_index("s") * NC + lax.axis_index("c")
        tokbase = wid * TOK_PER_W
        pltpu.sync_copy(pe_hbm.at[wid], pe_v)
        pltpu.sync_copy(po_hbm.at[wid], po_v)
        rows = [rows0, rows1]
        gd = [None, None]
        sd = [None, None]
        gd[0] = pltpu.async_copy(x_hbm.at[pl.ds(tokbase, CHUNK)], rows[0], gsem)
        for j in range(DCH):
            if j >= 1:
                sd[0].wait()
                sd[1].wait()
            if j + 1 < DCH:
                gd[(j + 1) % 2] = pltpu.async_copy(
                    x_hbm.at[pl.ds(tokbase + (j + 1) * CHUNK, CHUNK)],
                    rows[(j + 1) % 2], gsem)
            gd[j % 2].wait()
            sd[0] = pltpu.async_copy(rows[j % 2], xs_hbm.at[pe_v.at[j]], ssem)
            sd[1] = pltpu.async_copy(rows[j % 2], xs_hbm.at[po_v.at[j]], ssem)
        sd[0].wait()
        sd[1].wait()

    return body(x, pe, po)


def _sc_combine(ys, posk):
    @functools.partial(
        pl.kernel,
        out_type=jax.ShapeDtypeStruct((NSLOT, H), jnp.float32),
        mesh=_sc_mesh(),
        scratch_types=[
            pltpu.VMEM((CCH, CHUNK), jnp.int32),
            pltpu.VMEM((CHUNK, H), jnp.float32),
            pltpu.VMEM((CHUNK, H), jnp.float32),
            pltpu.SemaphoreType.DMA,
            pltpu.SemaphoreType.DMA,
        ],
    )
    def body(ys_hbm, posk_hbm, yg_hbm, pos_v, rows0, rows1, gsem, wsem):
        wid = lax.axis_index("s") * NC + lax.axis_index("c")
        base = wid * SLOT_PER_W
        pltpu.sync_copy(posk_hbm.at[wid], pos_v)
        rows = [rows0, rows1]
        gd = [None, None]
        wd = None
        gd[0] = pltpu.async_copy(ys_hbm.at[pos_v.at[0]], rows[0], gsem)
        for j in range(CCH):
            if j >= 1:
                wd.wait()
            if j + 1 < CCH:
                gd[(j + 1) % 2] = pltpu.async_copy(
                    ys_hbm.at[pos_v.at[j + 1]], rows[(j + 1) % 2], gsem)
            gd[j % 2].wait()
            wd = pltpu.async_copy(rows[j % 2],
                                  yg_hbm.at[pl.ds(base + j * CHUNK, CHUNK)], wsem)
        wd.wait()

    return body(ys, posk)


def _grouped_body(meta_ref, xs_ref, wg_ref, wu_ref, wd_ref, ys_ref):
    b = pl.program_id(0)

    @pl.when(b < meta_ref[NBLK])
    def _():
        xb = xs_ref[...]
        g = lax.dot_general(xb, wg_ref[0], (((1,), (1,)), ((), ())),
                            preferred_element_type=jnp.float32)
        u = lax.dot_general(xb, wu_ref[0], (((1,), (1,)), ((), ())),
                            preferred_element_type=jnp.float32)
        h = g * _sigmoid(g) * u
        ys_ref[...] = lax.dot_general(h, wd_ref[0], (((1,), (1,)), ((), ())),
                                      preferred_element_type=jnp.float32)


def _row_map(b, meta):
    return (jnp.where(b < meta[NBLK], b, meta[NBLK] - 1), 0)


def _grouped_mlp(xs, wg, wu, wd, meta):
    grid_spec = pltpu.PrefetchScalarGridSpec(
        num_scalar_prefetch=1,
        grid=(NBLK,),
        in_specs=[
            pl.BlockSpec((BT, H), _row_map),
            pl.BlockSpec((1, F, H), lambda b, meta: (meta[b], 0, 0)),
            pl.BlockSpec((1, F, H), lambda b, meta: (meta[b], 0, 0)),
            pl.BlockSpec((1, H, F), lambda b, meta: (meta[b], 0, 0)),
        ],
        out_specs=pl.BlockSpec((BT, H), _row_map),
    )
    return pl.pallas_call(
        _grouped_body,
        grid_spec=grid_spec,
        out_shape=jax.ShapeDtypeStruct((PAD, H), jnp.float32),
    )(meta, xs, wg, wu, wd)


def _shared_body(x_ref, sg_ref, su_ref, sd_ref, out_ref):
    xb = x_ref[...]
    g = lax.dot_general(xb, sg_ref[...], (((1,), (1,)), ((), ())),
                        preferred_element_type=jnp.float32)
    u = lax.dot_general(xb, su_ref[...], (((1,), (1,)), ((), ())),
                        preferred_element_type=jnp.float32)
    h = g * _sigmoid(g) * u
    out_ref[...] = lax.dot_general(h, sd_ref[...], (((1,), (1,)), ((), ())),
                                   preferred_element_type=jnp.float32)


def _shared(x, sg, su, sd):
    bt = 256
    return pl.pallas_call(
        _shared_body,
        grid=(T // bt,),
        in_specs=[
            pl.BlockSpec((bt, H), lambda i: (i, 0)),
            pl.BlockSpec((F, H), lambda i: (0, 0)),
            pl.BlockSpec((F, H), lambda i: (0, 0)),
            pl.BlockSpec((H, F), lambda i: (0, 0)),
        ],
        out_specs=pl.BlockSpec((bt, H), lambda i: (i, 0)),
        out_shape=jax.ShapeDtypeStruct((T, H), jnp.float32),
    )(x, sg, su, sd)


def _combine_body(yg_ref, w_ref, sh_ref, out_ref):
    w = w_ref[...]
    out_ref[...] = (sh_ref[...] + w[:, 0:1] * yg_ref[0] + w[:, 1:2] * yg_ref[1])


def _combine(yg, w, sh):
    bt = 256
    return pl.pallas_call(
        _combine_body,
        grid=(T // bt,),
        in_specs=[
            pl.BlockSpec((K, bt, H), lambda i: (0, i, 0)),
            pl.BlockSpec((bt, K), lambda i: (i, 0)),
            pl.BlockSpec((bt, H), lambda i: (i, 0)),
        ],
        out_specs=pl.BlockSpec((bt, H), lambda i: (i, 0)),
        out_shape=jax.ShapeDtypeStruct((T, H), jnp.float32),
    )(yg, w, sh)


def kernel(hidden_states, router_weight, expert_gate, expert_up, expert_down,
           shared_gate, shared_up, shared_down):
    bsz, seq, h = hidden_states.shape
    x = hidden_states.reshape(-1, h)

    topk_idx, topk_w = _router(x, router_weight)
    pos2d, meta2d = _metadata(topk_idx.reshape(_MR, _MC))
    meta = meta2d.reshape(2 * NBLK)
    pos_kt = pos2d.reshape(T, K).T
    pe = pos_kt[0].reshape(NW, DCH, CHUNK)
    po = pos_kt[1].reshape(NW, DCH, CHUNK)
    posk = pos_kt.reshape(NW, CCH, CHUNK)

    xs = _sc_dispatch(x, pe, po)
    sh = _shared(x, shared_gate, shared_up, shared_down)
    ys = _grouped_mlp(xs, expert_gate, expert_up, expert_down, meta)
    yg = _sc_combine(ys, posk)

    out = _combine(yg.reshape(K, T, H), topk_w, sh)
    return out.reshape(bsz, seq, h)


# speedup vs baseline: 1.0172x; 1.0172x over previous
import functools

import jax
import jax.numpy as jnp
from jax import lax
from jax.experimental import pallas as pl
from jax.experimental.pallas import tpu as pltpu
from jax.experimental.pallas import tpu_sc as plsc

E = 16
K = 2
H = 1024
F = 512
SCALE = 2.5
T = 4096
NSLOT = T * K
BT = 256
NBLK = 48
PAD = NBLK * BT

NC = 2
NS = 16
NW = NC * NS
CHUNK = 32
TOK_PER_W = T // NW
DCH = TOK_PER_W // CHUNK
SLOT_PER_W = NSLOT // NW
CCH = SLOT_PER_W // CHUNK


def _sigmoid(x):
    return 1.0 / (1.0 + jnp.exp(-x))


def _router_body(x_ref, wr_ref, idx_ref, w_ref):
    logits = lax.dot_general(x_ref[...], wr_ref[...],
                             (((1,), (1,)), ((), ())),
                             preferred_element_type=jnp.float32)
    scores = _sigmoid(logits)
    bt = scores.shape[0]
    iota = lax.broadcasted_iota(jnp.int32, (bt, E), 1)
    m1 = jnp.max(scores, axis=1, keepdims=True)
    i1 = jnp.min(jnp.where(scores == m1, iota, E), axis=1, keepdims=True)
    scores2 = jnp.where(iota == i1, -1.0, scores)
    m2 = jnp.max(scores2, axis=1, keepdims=True)
    i2 = jnp.min(jnp.where(scores2 == m2, iota, E), axis=1, keepdims=True)
    denom = m1 + m2 + 1e-20
    idx_ref[...] = jnp.concatenate([i1, i2], axis=1)
    w_ref[...] = jnp.concatenate([m1 / denom, m2 / denom], axis=1) * SCALE


def _router(x, wr):
    bt = 512
    return pl.pallas_call(
        _router_body,
        grid=(T // bt,),
        in_specs=[
            pl.BlockSpec((bt, H), lambda i: (i, 0)),
            pl.BlockSpec((E, H), lambda i: (0, 0)),
        ],
        out_specs=[
            pl.BlockSpec((bt, K), lambda i: (i, 0)),
            pl.BlockSpec((bt, K), lambda i: (i, 0)),
        ],
        out_shape=[
            jax.ShapeDtypeStruct((T, K), jnp.int32),
            jax.ShapeDtypeStruct((T, K), jnp.float32),
        ],
    )(x, wr)


_MR = 64
_MC = 128


def _metadata_body(e2d_ref, pos_ref, bex_ref):
    e2d = e2d_ref[...]
    r = lax.broadcasted_iota(jnp.int32, (_MC, _MC), 0)
    c = lax.broadcasted_iota(jnp.int32, (_MC, _MC), 1)
    upper_incl = (r <= c).astype(jnp.float32)
    r2 = lax.broadcasted_iota(jnp.int32, (_MR, _MR), 0)
    c2 = lax.broadcasted_iota(jnp.int32, (_MR, _MR), 1)
    lower_strict = (c2 < r2).astype(jnp.float32)

    counts = []
    for e in range(E):
        counts.append(jnp.sum((e2d == e).astype(jnp.float32)))
    pb = []
    acc = jnp.int32(0)
    blk_starts = []
    for e in range(E):
        blk_starts.append(acc)
        pb.append(acc * BT)
        nb = (counts[e].astype(jnp.int32) + (BT - 1)) // BT
        acc = acc + nb
    total_blocks = acc

    pos = jnp.zeros((_MR, _MC), jnp.float32)
    for e in range(E):
        m = (e2d == e).astype(jnp.float32)
        incl = lax.dot_general(m, upper_incl, (((1,), (0,)), ((), ())),
                               preferred_element_type=jnp.float32)
        tot = jnp.sum(m, axis=1, keepdims=True)
        ro = lax.dot_general(lower_strict, tot, (((1,), (0,)), ((), ())),
                             preferred_element_type=jnp.float32)
        rank = ro + incl - 1.0
        pos = pos + m * (pb[e].astype(jnp.float32) + rank)
    pos_ref[...] = pos.astype(jnp.int32)

    b_iota = lax.broadcasted_iota(jnp.int32, (1, NBLK), 1)
    bex = jnp.zeros((1, NBLK), jnp.int32)
    last_e = jnp.int32(-1)
    for e in range(E):
        bex = bex + (b_iota >= blk_starts[e]).astype(jnp.int32)
        last_e = last_e + (blk_starts[e] < total_blocks).astype(jnp.int32)
    bex = bex - 1
    bex_ref[0:1, :] = jnp.where(b_iota >= total_blocks, last_e, bex)
    bex_ref[1:2, :] = jnp.full((1, NBLK), 1, jnp.int32) * total_blocks


def _metadata(e2d):
    return pl.pallas_call(
        _metadata_body,
        in_specs=[pl.BlockSpec((_MR, _MC), lambda: (0, 0))],
        out_specs=[
            pl.BlockSpec((_MR, _MC), lambda: (0, 0)),
            pl.BlockSpec((2, NBLK), lambda: (0, 0)),
        ],
        out_shape=[
            jax.ShapeDtypeStruct((_MR, _MC), jnp.int32),
            jax.ShapeDtypeStruct((2, NBLK), jnp.int32),
        ],
    )(e2d)


def _sc_mesh():
    return plsc.VectorSubcoreMesh(core_axis_name="c", subcore_axis_name="s",
                                  num_cores=NC, num_subcores=NS)


def _sc_dispatch(x, pe, po):
    @functools.partial(
        pl.kernel,
        out_type=jax.ShapeDtypeStruct((PAD, H), jnp.float32),
        mesh=_sc_mesh(),
        scratch_types=[
            pltpu.VMEM((DCH, CHUNK), jnp.int32),
            pltpu.VMEM((DCH, CHUNK), jnp.int32),
            pltpu.VMEM((CHUNK, H), jnp.float32),
            pltpu.VMEM((CHUNK, H), jnp.float32),
            pltpu.SemaphoreType.DMA,
            pltpu.SemaphoreType.DMA,
        ],
    )
    def body(x_hbm, pe_hbm, po_hbm, xs_hbm, pe_v, po_v, rows0, rows1, gsem, ssem):
        wid = lax.axis_index("s") * NC + lax.axis_index("c")
        tokbase = wid * TOK_PER_W
        pltpu.sync_copy(pe_hbm.at[wid], pe_v)
        pltpu.sync_copy(po_hbm.at[wid], po_v)
        rows = [rows0, rows1]
        gd = [None, None]
        sd = [None, None]
        gd[0] = pltpu.async_copy(x_hbm.at[pl.ds(tokbase, CHUNK)], rows[0], gsem)
        for j in range(DCH):
            if j >= 1:
                sd[0].wait()
                sd[1].wait()
            if j + 1 < DCH:
                gd[(j + 1) % 2] = pltpu.async_copy(
                    x_hbm.at[pl.ds(tokbase + (j + 1) * CHUNK, CHUNK)],
                    rows[(j + 1) % 2], gsem)
            gd[j % 2].wait()
            sd[0] = pltpu.async_copy(rows[j % 2], xs_hbm.at[pe_v.at[j]], ssem)
            sd[1] = pltpu.async_copy(rows[j % 2], xs_hbm.at[po_v.at[j]], ssem)
        sd[0].wait()
        sd[1].wait()

    return body(x, pe, po)


def _sc_combine(ys, posk):
    @functools.partial(
        pl.kernel,
        out_type=jax.ShapeDtypeStruct((NSLOT, H), jnp.float32),
        mesh=_sc_mesh(),
        scratch_types=[
            pltpu.VMEM((CCH, CHUNK), jnp.int32),
            pltpu.VMEM((CHUNK, H), jnp.float32),
            pltpu.VMEM((CHUNK, H), jnp.float32),
            pltpu.SemaphoreType.DMA,
            pltpu.SemaphoreType.DMA,
        ],
    )
    def body(ys_hbm, posk_hbm, yg_hbm, pos_v, rows0, rows1, gsem, wsem):
        wid = lax.axis_index("s") * NC + lax.axis_index("c")
        base = wid * SLOT_PER_W
        pltpu.sync_copy(posk_hbm.at[wid], pos_v)
        rows = [rows0, rows1]
        gd = [None, None]
        wd = None
        gd[0] = pltpu.async_copy(ys_hbm.at[pos_v.at[0]], rows[0], gsem)
        for j in range(CCH):
            if j >= 1:
                wd.wait()
            if j + 1 < CCH:
                gd[(j + 1) % 2] = pltpu.async_copy(
                    ys_hbm.at[pos_v.at[j + 1]], rows[(j + 1) % 2], gsem)
            gd[j % 2].wait()
            wd = pltpu.async_copy(rows[j % 2],
                                  yg_hbm.at[pl.ds(base + j * CHUNK, CHUNK)], wsem)
        wd.wait()

    return body(ys, posk)


def _grouped_body(meta_ref, xs_ref, wg_ref, wu_ref, wd_ref, ys_ref):
    b = pl.program_id(0)

    @pl.when(b < meta_ref[NBLK])
    def _():
        xb = xs_ref[...]
        g = lax.dot_general(xb, wg_ref[0], (((1,), (1,)), ((), ())),
                            preferred_element_type=jnp.float32)
        u = lax.dot_general(xb, wu_ref[0], (((1,), (1,)), ((), ())),
                            preferred_element_type=jnp.float32)
        h = g * _sigmoid(g) * u
        ys_ref[...] = lax.dot_general(h, wd_ref[0], (((1,), (1,)), ((), ())),
                                      preferred_element_type=jnp.float32)


def _row_map(b, meta):
    return (jnp.where(b < meta[NBLK], b, meta[NBLK] - 1), 0)


def _grouped_mlp(xs, wg, wu, wd, meta):
    grid_spec = pltpu.PrefetchScalarGridSpec(
        num_scalar_prefetch=1,
        grid=(NBLK,),
        in_specs=[
            pl.BlockSpec((BT, H), _row_map),
            pl.BlockSpec((1, F, H), lambda b, meta: (meta[b], 0, 0)),
            pl.BlockSpec((1, F, H), lambda b, meta: (meta[b], 0, 0)),
            pl.BlockSpec((1, H, F), lambda b, meta: (meta[b], 0, 0)),
        ],
        out_specs=pl.BlockSpec((BT, H), _row_map),
    )
    return pl.pallas_call(
        _grouped_body,
        grid_spec=grid_spec,
        out_shape=jax.ShapeDtypeStruct((PAD, H), jnp.float32),
    )(meta, xs, wg, wu, wd)


def _shared_body(x_ref, sg_ref, su_ref, sd_ref, out_ref):
    xb = x_ref[...]
    g = lax.dot_general(xb, sg_ref[...], (((1,), (1,)), ((), ())),
                        preferred_element_type=jnp.float32)
    u = lax.dot_general(xb, su_ref[...], (((1,), (1,)), ((), ())),
                        preferred_element_type=jnp.float32)
    h = g * _sigmoid(g) * u
    out_ref[...] = lax.dot_general(h, sd_ref[...], (((1,), (1,)), ((), ())),
                                   preferred_element_type=jnp.float32)


def _shared(x, sg, su, sd):
    bt = 256
    return pl.pallas_call(
        _shared_body,
        grid=(T // bt,),
        in_specs=[
            pl.BlockSpec((bt, H), lambda i: (i, 0)),
            pl.BlockSpec((F, H), lambda i: (0, 0)),
            pl.BlockSpec((F, H), lambda i: (0, 0)),
            pl.BlockSpec((H, F), lambda i: (0, 0)),
        ],
        out_specs=pl.BlockSpec((bt, H), lambda i: (i, 0)),
        out_shape=jax.ShapeDtypeStruct((T, H), jnp.float32),
    )(x, sg, su, sd)


def _combine_body(yg_ref, w_ref, sh_ref, out_ref):
    w = w_ref[...]
    out_ref[...] = (sh_ref[...] + w[:, 0:1] * yg_ref[0] + w[:, 1:2] * yg_ref[1])


def _combine(yg, w, sh):
    bt = 256
    return pl.pallas_call(
        _combine_body,
        grid=(T // bt,),
        in_specs=[
            pl.BlockSpec((K, bt, H), lambda i: (0, i, 0)),
            pl.BlockSpec((bt, K), lambda i: (i, 0)),
            pl.BlockSpec((bt, H), lambda i: (i, 0)),
        ],
        out_specs=pl.BlockSpec((bt, H), lambda i: (i, 0)),
        out_shape=jax.ShapeDtypeStruct((T, H), jnp.float32),
    )(yg, w, sh)


def _final_body(x_ref, yg_ref, w_ref, sg_ref, su_ref, sd_ref, out_ref):
    xb = x_ref[...]
    g = lax.dot_general(xb, sg_ref[...], (((1,), (1,)), ((), ())),
                        preferred_element_type=jnp.float32)
    u = lax.dot_general(xb, su_ref[...], (((1,), (1,)), ((), ())),
                        preferred_element_type=jnp.float32)
    h = g * _sigmoid(g) * u
    sh = lax.dot_general(h, sd_ref[...], (((1,), (1,)), ((), ())),
                         preferred_element_type=jnp.float32)
    w = w_ref[...]
    out_ref[...] = sh + w[:, 0:1] * yg_ref[0] + w[:, 1:2] * yg_ref[1]


def _final(x, yg, w, sg, su, sd):
    bt = 256
    return pl.pallas_call(
        _final_body,
        grid=(T // bt,),
        in_specs=[
            pl.BlockSpec((bt, H), lambda i: (i, 0)),
            pl.BlockSpec((K, bt, H), lambda i: (0, i, 0)),
            pl.BlockSpec((bt, K), lambda i: (i, 0)),
            pl.BlockSpec((F, H), lambda i: (0, 0)),
            pl.BlockSpec((F, H), lambda i: (0, 0)),
            pl.BlockSpec((H, F), lambda i: (0, 0)),
        ],
        out_specs=pl.BlockSpec((bt, H), lambda i: (i, 0)),
        out_shape=jax.ShapeDtypeStruct((T, H), jnp.float32),
    )(x, yg, w, sg, su, sd)


def kernel(hidden_states, router_weight, expert_gate, expert_up, expert_down,
           shared_gate, shared_up, shared_down):
    bsz, seq, h = hidden_states.shape
    x = hidden_states.reshape(-1, h)

    topk_idx, topk_w = _router(x, router_weight)
    pos2d, meta2d = _metadata(topk_idx.reshape(_MR, _MC))
    meta = meta2d.reshape(2 * NBLK)
    pos_kt = pos2d.reshape(T, K).T
    pe = pos_kt[0].reshape(NW, DCH, CHUNK)
    po = pos_kt[1].reshape(NW, DCH, CHUNK)
    posk = pos_kt.reshape(NW, CCH, CHUNK)

    xs = _sc_dispatch(x, pe, po)
    ys = _grouped_mlp(xs, expert_gate, expert_up, expert_down, meta)
    yg = _sc_combine(ys, posk)

    out = _final(x, yg.reshape(K, T, H), topk_w,
                 shared_gate, shared_up, shared_down)
    return out.reshape(bsz, seq, h)


# speedup vs baseline: 1.0176x; 1.0004x over previous
import functools

import jax
import jax.numpy as jnp
from jax import lax
from jax.experimental import pallas as pl
from jax.experimental.pallas import tpu as pltpu
from jax.experimental.pallas import tpu_sc as plsc

E = 16
K = 2
H = 1024
F = 512
SCALE = 2.5
T = 4096
NSLOT = T * K
BT = 256
NBLK = 48
PAD = NBLK * BT

NC = 2
NS = 16
NW = NC * NS
CHUNK = 32
TOK_PER_W = T // NW
DCH = TOK_PER_W // CHUNK
SLOT_PER_W = NSLOT // NW
CCH = SLOT_PER_W // CHUNK


def _sigmoid(x):
    return 1.0 / (1.0 + jnp.exp(-x))


def _router_body(x_ref, wr_ref, idx_ref, w_ref):
    logits = lax.dot_general(x_ref[...], wr_ref[...],
                             (((1,), (1,)), ((), ())),
                             preferred_element_type=jnp.float32)
    scores = _sigmoid(logits)
    bt = scores.shape[0]
    iota = lax.broadcasted_iota(jnp.int32, (bt, E), 1)
    m1 = jnp.max(scores, axis=1, keepdims=True)
    i1 = jnp.min(jnp.where(scores == m1, iota, E), axis=1, keepdims=True)
    scores2 = jnp.where(iota == i1, -1.0, scores)
    m2 = jnp.max(scores2, axis=1, keepdims=True)
    i2 = jnp.min(jnp.where(scores2 == m2, iota, E), axis=1, keepdims=True)
    denom = m1 + m2 + 1e-20
    idx_ref[...] = jnp.concatenate([i1, i2], axis=1)
    w_ref[...] = jnp.concatenate([m1 / denom, m2 / denom], axis=1) * SCALE


def _router(x, wr):
    bt = 512
    return pl.pallas_call(
        _router_body,
        grid=(T // bt,),
        in_specs=[
            pl.BlockSpec((bt, H), lambda i: (i, 0)),
            pl.BlockSpec((E, H), lambda i: (0, 0)),
        ],
        out_specs=[
            pl.BlockSpec((bt, K), lambda i: (i, 0)),
            pl.BlockSpec((bt, K), lambda i: (i, 0)),
        ],
        out_shape=[
            jax.ShapeDtypeStruct((T, K), jnp.int32),
            jax.ShapeDtypeStruct((T, K), jnp.float32),
        ],
    )(x, wr)


_MR = 64
_MC = 128


def _metadata_body(e2d_ref, pos_ref, bex_ref):
    e2d = e2d_ref[...]
    r = lax.broadcasted_iota(jnp.int32, (_MC, _MC), 0)
    c = lax.broadcasted_iota(jnp.int32, (_MC, _MC), 1)
    upper_incl = (r <= c).astype(jnp.float32)
    r2 = lax.broadcasted_iota(jnp.int32, (_MR, _MR), 0)
    c2 = lax.broadcasted_iota(jnp.int32, (_MR, _MR), 1)
    lower_strict = (c2 < r2).astype(jnp.float32)

    counts = []
    for e in range(E):
        counts.append(jnp.sum((e2d == e).astype(jnp.float32)))
    pb = []
    acc = jnp.int32(0)
    blk_starts = []
    for e in range(E):
        blk_starts.append(acc)
        pb.append(acc * BT)
        nb = (counts[e].astype(jnp.int32) + (BT - 1)) // BT
        acc = acc + nb
    total_blocks = acc

    pos = jnp.zeros((_MR, _MC), jnp.float32)
    for e in range(E):
        m = (e2d == e).astype(jnp.float32)
        incl = lax.dot_general(m, upper_incl, (((1,), (0,)), ((), ())),
                               preferred_element_type=jnp.float32)
        tot = jnp.sum(m, axis=1, keepdims=True)
        ro = lax.dot_general(lower_strict, tot, (((1,), (0,)), ((), ())),
                             preferred_element_type=jnp.float32)
        rank = ro + incl - 1.0
        pos = pos + m * (pb[e].astype(jnp.float32) + rank)
    pos_ref[...] = pos.astype(jnp.int32)

    b_iota = lax.broadcasted_iota(jnp.int32, (1, NBLK), 1)
    bex = jnp.zeros((1, NBLK), jnp.int32)
    last_e = jnp.int32(-1)
    for e in range(E):
        bex = bex + (b_iota >= blk_starts[e]).astype(jnp.int32)
        last_e = last_e + (blk_starts[e] < total_blocks).astype(jnp.int32)
    bex = bex - 1
    bex_ref[0:1, :] = jnp.where(b_iota >= total_blocks, last_e, bex)
    bex_ref[1:2, :] = jnp.full((1, NBLK), 1, jnp.int32) * total_blocks


def _metadata(e2d):
    return pl.pallas_call(
        _metadata_body,
        in_specs=[pl.BlockSpec((_MR, _MC), lambda: (0, 0))],
        out_specs=[
            pl.BlockSpec((_MR, _MC), lambda: (0, 0)),
            pl.BlockSpec((2, NBLK), lambda: (0, 0)),
        ],
        out_shape=[
            jax.ShapeDtypeStruct((_MR, _MC), jnp.int32),
            jax.ShapeDtypeStruct((2, NBLK), jnp.int32),
        ],
    )(e2d)


def _sc_mesh():
    return plsc.VectorSubcoreMesh(core_axis_name="c", subcore_axis_name="s",
                                  num_cores=NC, num_subcores=NS)


def _sc_dispatch(x, pe, po):
    @functools.partial(
        pl.kernel,
        out_type=jax.ShapeDtypeStruct((PAD, H), jnp.float32),
        mesh=_sc_mesh(),
        scratch_types=[
            pltpu.VMEM((DCH, CHUNK), jnp.int32),
            pltpu.VMEM((DCH, CHUNK), jnp.int32),
            pltpu.VMEM((CHUNK, H), jnp.float32),
            pltpu.VMEM((CHUNK, H), jnp.float32),
            pltpu.SemaphoreType.DMA,
            pltpu.SemaphoreType.DMA,
        ],
    )
    def body(x_hbm, pe_hbm, po_hbm, xs_hbm, pe_v, po_v, rows0, rows1, gsem, ssem):
        wid = lax.axis_index("s") * NC + lax.axis_index("c")
        tokbase = wid * TOK_PER_W
        pltpu.sync_copy(pe_hbm.at[wid], pe_v)
        pltpu.sync_copy(po_hbm.at[wid], po_v)
        rows = [rows0, rows1]
        gd = [None, None]
        sd = [None, None]
        gd[0] = pltpu.async_copy(x_hbm.at[pl.ds(tokbase, CHUNK)], rows[0], gsem)
        for j in range(DCH):
            if j >= 1:
                sd[0].wait()
                sd[1].wait()
            if j + 1 < DCH:
                gd[(j + 1) % 2] = pltpu.async_copy(
                    x_hbm.at[pl.ds(tokbase + (j + 1) * CHUNK, CHUNK)],
                    rows[(j + 1) % 2], gsem)
            gd[j % 2].wait()
            sd[0] = pltpu.async_copy(rows[j % 2], xs_hbm.at[pe_v.at[j]], ssem)
            sd[1] = pltpu.async_copy(rows[j % 2], xs_hbm.at[po_v.at[j]], ssem)
        sd[0].wait()
        sd[1].wait()

    return body(x, pe, po)


def _sc_combine(ys, posk):
    @functools.partial(
        pl.kernel,
        out_type=jax.ShapeDtypeStruct((NSLOT, H), jnp.float32),
        mesh=_sc_mesh(),
        scratch_types=[
            pltpu.VMEM((CCH, CHUNK), jnp.int32),
            pltpu.VMEM((CHUNK, H), jnp.float32),
            pltpu.VMEM((CHUNK, H), jnp.float32),
            pltpu.SemaphoreType.DMA,
            pltpu.SemaphoreType.DMA,
        ],
    )
    def body(ys_hbm, posk_hbm, yg_hbm, pos_v, rows0, rows1, gsem, wsem):
        wid = lax.axis_index("s") * NC + lax.axis_index("c")
        base = wid * SLOT_PER_W
        pltpu.sync_copy(posk_hbm.at[wid], pos_v)
        rows = [rows0, rows1]
        gd = [None, None]
        wd = None
        gd[0] = pltpu.async_copy(ys_hbm.at[pos_v.at[0]], rows[0], gsem)
        for j in range(CCH):
            if j >= 1:
                wd.wait()
            if j + 1 < CCH:
                gd[(j + 1) % 2] = pltpu.async_copy(
                    ys_hbm.at[pos_v.at[j + 1]], rows[(j + 1) % 2], gsem)
            gd[j % 2].wait()
            wd = pltpu.async_copy(rows[j % 2],
                                  yg_hbm.at[pl.ds(base + j * CHUNK, CHUNK)], wsem)
        wd.wait()

    return body(ys, posk)


def _grouped_body(meta_ref, xs_ref, wg_ref, wu_ref, wd_ref, ys_ref):
    b = pl.program_id(0)

    @pl.when(b < meta_ref[NBLK])
    def _():
        xb = xs_ref[...]
        g = lax.dot_general(xb, wg_ref[0], (((1,), (1,)), ((), ())),
                            preferred_element_type=jnp.float32)
        u = lax.dot_general(xb, wu_ref[0], (((1,), (1,)), ((), ())),
                            preferred_element_type=jnp.float32)
        h = g * _sigmoid(g) * u
        ys_ref[...] = lax.dot_general(h, wd_ref[0], (((1,), (1,)), ((), ())),
                                      preferred_element_type=jnp.float32)


def _row_map(b, meta):
    return (jnp.where(b < meta[NBLK], b, meta[NBLK] - 1), 0)


def _grouped_mlp(xs, wg, wu, wd, meta):
    grid_spec = pltpu.PrefetchScalarGridSpec(
        num_scalar_prefetch=1,
        grid=(NBLK,),
        in_specs=[
            pl.BlockSpec((BT, H), _row_map),
            pl.BlockSpec((1, F, H), lambda b, meta: (meta[b], 0, 0)),
            pl.BlockSpec((1, F, H), lambda b, meta: (meta[b], 0, 0)),
            pl.BlockSpec((1, H, F), lambda b, meta: (meta[b], 0, 0)),
        ],
        out_specs=pl.BlockSpec((BT, H), _row_map),
    )
    return pl.pallas_call(
        _grouped_body,
        grid_spec=grid_spec,
        out_shape=jax.ShapeDtypeStruct((PAD, H), jnp.float32),
    )(meta, xs, wg, wu, wd)


def _final_body(x_ref, yg_ref, w_ref, sg_ref, su_ref, sd_ref, out_ref):
    xb = x_ref[...]
    g = lax.dot_general(xb, sg_ref[...], (((1,), (1,)), ((), ())),
                        preferred_element_type=jnp.float32)
    u = lax.dot_general(xb, su_ref[...], (((1,), (1,)), ((), ())),
                        preferred_element_type=jnp.float32)
    h = g * _sigmoid(g) * u
    sh = lax.dot_general(h, sd_ref[...], (((1,), (1,)), ((), ())),
                         preferred_element_type=jnp.float32)
    w = w_ref[...]
    out_ref[...] = sh + w[:, 0:1] * yg_ref[0] + w[:, 1:2] * yg_ref[1]


def _final(x, yg, w, sg, su, sd):
    bt = 256
    return pl.pallas_call(
        _final_body,
        grid=(T // bt,),
        in_specs=[
            pl.BlockSpec((bt, H), lambda i: (i, 0)),
            pl.BlockSpec((K, bt, H), lambda i: (0, i, 0)),
            pl.BlockSpec((bt, K), lambda i: (i, 0)),
            pl.BlockSpec((F, H), lambda i: (0, 0)),
            pl.BlockSpec((F, H), lambda i: (0, 0)),
            pl.BlockSpec((H, F), lambda i: (0, 0)),
        ],
        out_specs=pl.BlockSpec((bt, H), lambda i: (i, 0)),
        out_shape=jax.ShapeDtypeStruct((T, H), jnp.float32),
    )(x, yg, w, sg, su, sd)


def kernel(hidden_states, router_weight, expert_gate, expert_up, expert_down,
           shared_gate, shared_up, shared_down):
    bsz, seq, h = hidden_states.shape
    x = hidden_states.reshape(-1, h)

    topk_idx, topk_w = _router(x, router_weight)
    pos2d, meta2d = _metadata(topk_idx.reshape(_MR, _MC))
    meta = meta2d.reshape(2 * NBLK)
    pos_kt = pos2d.reshape(T, K).T
    pe = pos_kt[0].reshape(NW, DCH, CHUNK)
    po = pos_kt[1].reshape(NW, DCH, CHUNK)
    posk = pos_kt.reshape(NW, CCH, CHUNK)

    xs = _sc_dispatch(x, pe, po)
    ys = _grouped_mlp(xs, expert_gate, expert_up, expert_down, meta)
    yg = _sc_combine(ys, posk)

    out = _final(x, yg.reshape(K, T, H), topk_w,
                 shared_gate, shared_up, shared_down)
    return out.reshape(bsz, seq, h)


# speedup vs baseline: 1.0204x; 1.0027x over previous
import functools

import jax
import jax.numpy as jnp
from jax import lax
from jax.experimental import pallas as pl
from jax.experimental.pallas import tpu as pltpu
from jax.experimental.pallas import tpu_sc as plsc

E = 16
K = 2
H = 1024
F = 512
SCALE = 2.5
T = 4096
NSLOT = T * K
BT = 256
NBLK = 48
PAD = NBLK * BT

NC = 2
NS = 16
NW = NC * NS
CHUNK = 32
TOK_PER_W = T // NW
DCH = TOK_PER_W // CHUNK
SLOT_PER_W = NSLOT // NW
CCH = SLOT_PER_W // CHUNK


def _sigmoid(x):
    return 1.0 / (1.0 + jnp.exp(-x))


def _router_body(x_ref, wr_ref, idx_ref, w_ref):
    logits = lax.dot_general(x_ref[...], wr_ref[...],
                             (((1,), (1,)), ((), ())),
                             preferred_element_type=jnp.float32)
    scores = _sigmoid(logits)
    bt = scores.shape[0]
    iota = lax.broadcasted_iota(jnp.int32, (bt, E), 1)
    m1 = jnp.max(scores, axis=1, keepdims=True)
    i1 = jnp.min(jnp.where(scores == m1, iota, E), axis=1, keepdims=True)
    scores2 = jnp.where(iota == i1, -1.0, scores)
    m2 = jnp.max(scores2, axis=1, keepdims=True)
    i2 = jnp.min(jnp.where(scores2 == m2, iota, E), axis=1, keepdims=True)
    denom = m1 + m2 + 1e-20
    idx_ref[...] = jnp.concatenate([i1, i2], axis=1)
    w_ref[...] = jnp.concatenate([m1 / denom, m2 / denom], axis=1) * SCALE


def _router(x, wr):
    bt = 512
    return pl.pallas_call(
        _router_body,
        grid=(T // bt,),
        in_specs=[
            pl.BlockSpec((bt, H), lambda i: (i, 0)),
            pl.BlockSpec((E, H), lambda i: (0, 0)),
        ],
        out_specs=[
            pl.BlockSpec((bt, K), lambda i: (i, 0)),
            pl.BlockSpec((bt, K), lambda i: (i, 0)),
        ],
        out_shape=[
            jax.ShapeDtypeStruct((T, K), jnp.int32),
            jax.ShapeDtypeStruct((T, K), jnp.float32),
        ],
    )(x, wr)


_MR = 64
_MC = 128


def _metadata_body(e2d_ref, pos_ref, bex_ref):
    e2d = e2d_ref[...]
    r = lax.broadcasted_iota(jnp.int32, (_MC, _MC), 0)
    c = lax.broadcasted_iota(jnp.int32, (_MC, _MC), 1)
    upper_incl = (r <= c).astype(jnp.float32)
    r2 = lax.broadcasted_iota(jnp.int32, (_MR, _MR), 0)
    c2 = lax.broadcasted_iota(jnp.int32, (_MR, _MR), 1)
    lower_strict = (c2 < r2).astype(jnp.float32)

    counts = []
    for e in range(E):
        counts.append(jnp.sum((e2d == e).astype(jnp.float32)))
    pb = []
    acc = jnp.int32(0)
    blk_starts = []
    for e in range(E):
        blk_starts.append(acc)
        pb.append(acc * BT)
        nb = (counts[e].astype(jnp.int32) + (BT - 1)) // BT
        acc = acc + nb
    total_blocks = acc

    pos = jnp.zeros((_MR, _MC), jnp.float32)
    for e in range(E):
        m = (e2d == e).astype(jnp.float32)
        incl = lax.dot_general(m, upper_incl, (((1,), (0,)), ((), ())),
                               preferred_element_type=jnp.float32)
        tot = jnp.sum(m, axis=1, keepdims=True)
        ro = lax.dot_general(lower_strict, tot, (((1,), (0,)), ((), ())),
                             preferred_element_type=jnp.float32)
        rank = ro + incl - 1.0
        pos = pos + m * (pb[e].astype(jnp.float32) + rank)
    pos_ref[...] = pos.astype(jnp.int32)

    b_iota = lax.broadcasted_iota(jnp.int32, (1, NBLK), 1)
    bex = jnp.zeros((1, NBLK), jnp.int32)
    last_e = jnp.int32(-1)
    for e in range(E):
        bex = bex + (b_iota >= blk_starts[e]).astype(jnp.int32)
        last_e = last_e + (blk_starts[e] < total_blocks).astype(jnp.int32)
    bex = bex - 1
    bex_ref[0:1, :] = jnp.where(b_iota >= total_blocks, last_e, bex)
    bex_ref[1:2, :] = jnp.full((1, NBLK), 1, jnp.int32) * total_blocks


def _metadata(e2d):
    return pl.pallas_call(
        _metadata_body,
        in_specs=[pl.BlockSpec((_MR, _MC), lambda: (0, 0))],
        out_specs=[
            pl.BlockSpec((_MR, _MC), lambda: (0, 0)),
            pl.BlockSpec((2, NBLK), lambda: (0, 0)),
        ],
        out_shape=[
            jax.ShapeDtypeStruct((_MR, _MC), jnp.int32),
            jax.ShapeDtypeStruct((2, NBLK), jnp.int32),
        ],
    )(e2d)


def _sc_mesh():
    return plsc.VectorSubcoreMesh(core_axis_name="c", subcore_axis_name="s",
                                  num_cores=NC, num_subcores=NS)


def _sc_dispatch(x, pe, po):
    @functools.partial(
        pl.kernel,
        out_type=jax.ShapeDtypeStruct((PAD, H), jnp.float32),
        mesh=_sc_mesh(),
        scratch_types=[
            pltpu.VMEM((DCH, CHUNK), jnp.int32),
            pltpu.VMEM((DCH, CHUNK), jnp.int32),
            pltpu.VMEM((CHUNK, H), jnp.float32),
            pltpu.VMEM((CHUNK, H), jnp.float32),
            pltpu.VMEM((CHUNK, H), jnp.float32),
            pltpu.SemaphoreType.DMA,
            pltpu.SemaphoreType.DMA,
        ],
    )
    def body(x_hbm, pe_hbm, po_hbm, xs_hbm, pe_v, po_v, rows0, rows1, rows2,
             gsem, ssem):
        wid = lax.axis_index("s") * NC + lax.axis_index("c")
        tokbase = wid * TOK_PER_W
        pltpu.sync_copy(pe_hbm.at[wid], pe_v)
        pltpu.sync_copy(po_hbm.at[wid], po_v)
        rows = [rows0, rows1, rows2]
        gd = [None, None, None]
        sd = [None] * DCH
        gd[0] = pltpu.async_copy(x_hbm.at[pl.ds(tokbase, CHUNK)], rows[0], gsem)
        gd[1] = pltpu.async_copy(x_hbm.at[pl.ds(tokbase + CHUNK, CHUNK)],
                                 rows[1], gsem)
        for j in range(DCH):
            if j + 2 < DCH:
                if j >= 1:
                    sd[j - 1][0].wait()
                    sd[j - 1][1].wait()
                gd[(j + 2) % 3] = pltpu.async_copy(
                    x_hbm.at[pl.ds(tokbase + (j + 2) * CHUNK, CHUNK)],
                    rows[(j + 2) % 3], gsem)
            gd[j % 3].wait()
            sd[j] = (pltpu.async_copy(rows[j % 3], xs_hbm.at[pe_v.at[j]], ssem),
                     pltpu.async_copy(rows[j % 3], xs_hbm.at[po_v.at[j]], ssem))
        for j in range(DCH):
            if sd[j] is not None and j >= DCH - 3:
                sd[j][0].wait()
                sd[j][1].wait()

    return body(x, pe, po)


def _sc_combine(ys, posk):
    @functools.partial(
        pl.kernel,
        out_type=jax.ShapeDtypeStruct((NSLOT, H), jnp.float32),
        mesh=_sc_mesh(),
        scratch_types=[
            pltpu.VMEM((CCH, CHUNK), jnp.int32),
            pltpu.VMEM((CHUNK, H), jnp.float32),
            pltpu.VMEM((CHUNK, H), jnp.float32),
            pltpu.VMEM((CHUNK, H), jnp.float32),
            pltpu.SemaphoreType.DMA,
            pltpu.SemaphoreType.DMA,
        ],
    )
    def body(ys_hbm, posk_hbm, yg_hbm, pos_v, rows0, rows1, rows2, gsem, wsem):
        wid = lax.axis_index("s") * NC + lax.axis_index("c")
        base = wid * SLOT_PER_W
        pltpu.sync_copy(posk_hbm.at[wid], pos_v)
        rows = [rows0, rows1, rows2]
        gd = [None, None, None]
        wd = [None] * CCH
        gd[0] = pltpu.async_copy(ys_hbm.at[pos_v.at[0]], rows[0], gsem)
        gd[1] = pltpu.async_copy(ys_hbm.at[pos_v.at[1]], rows[1], gsem)
        for j in range(CCH):
            if j + 2 < CCH:
                if j >= 1:
                    wd[j - 1].wait()
                gd[(j + 2) % 3] = pltpu.async_copy(
                    ys_hbm.at[pos_v.at[j + 2]], rows[(j + 2) % 3], gsem)
            gd[j % 3].wait()
            wd[j] = pltpu.async_copy(rows[j % 3],
                                     yg_hbm.at[pl.ds(base + j * CHUNK, CHUNK)],
                                     wsem)
        for j in range(CCH - 3, CCH):
            wd[j].wait()

    return body(ys, posk)


def _grouped_body(meta_ref, xs_ref, wg_ref, wu_ref, wd_ref, ys_ref):
    b = pl.program_id(0)

    @pl.when(b < meta_ref[NBLK])
    def _():
        xb = xs_ref[...]
        g = lax.dot_general(xb, wg_ref[0], (((1,), (1,)), ((), ())),
                            preferred_element_type=jnp.float32)
        u = lax.dot_general(xb, wu_ref[0], (((1,), (1,)), ((), ())),
                            preferred_element_type=jnp.float32)
        h = g * _sigmoid(g) * u
        ys_ref[...] = lax.dot_general(h, wd_ref[0], (((1,), (1,)), ((), ())),
                                      preferred_element_type=jnp.float32)


def _row_map(b, meta):
    return (jnp.where(b < meta[NBLK], b, meta[NBLK] - 1), 0)


def _grouped_mlp(xs, wg, wu, wd, meta):
    grid_spec = pltpu.PrefetchScalarGridSpec(
        num_scalar_prefetch=1,
        grid=(NBLK,),
        in_specs=[
            pl.BlockSpec((BT, H), _row_map),
            pl.BlockSpec((1, F, H), lambda b, meta: (meta[b], 0, 0)),
            pl.BlockSpec((1, F, H), lambda b, meta: (meta[b], 0, 0)),
            pl.BlockSpec((1, H, F), lambda b, meta: (meta[b], 0, 0)),
        ],
        out_specs=pl.BlockSpec((BT, H), _row_map),
    )
    return pl.pallas_call(
        _grouped_body,
        grid_spec=grid_spec,
        out_shape=jax.ShapeDtypeStruct((PAD, H), jnp.float32),
    )(meta, xs, wg, wu, wd)


def _final_body(x_ref, yg_ref, w_ref, sg_ref, su_ref, sd_ref, out_ref):
    xb = x_ref[...]
    g = lax.dot_general(xb, sg_ref[...], (((1,), (1,)), ((), ())),
                        preferred_element_type=jnp.float32)
    u = lax.dot_general(xb, su_ref[...], (((1,), (1,)), ((), ())),
                        preferred_element_type=jnp.float32)
    h = g * _sigmoid(g) * u
    sh = lax.dot_general(h, sd_ref[...], (((1,), (1,)), ((), ())),
                         preferred_element_type=jnp.float32)
    w = w_ref[...]
    out_ref[...] = sh + w[:, 0:1] * yg_ref[0] + w[:, 1:2] * yg_ref[1]


def _final(x, yg, w, sg, su, sd):
    bt = 256
    return pl.pallas_call(
        _final_body,
        grid=(T // bt,),
        in_specs=[
            pl.BlockSpec((bt, H), lambda i: (i, 0)),
            pl.BlockSpec((K, bt, H), lambda i: (0, i, 0)),
            pl.BlockSpec((bt, K), lambda i: (i, 0)),
            pl.BlockSpec((F, H), lambda i: (0, 0)),
            pl.BlockSpec((F, H), lambda i: (0, 0)),
            pl.BlockSpec((H, F), lambda i: (0, 0)),
        ],
        out_specs=pl.BlockSpec((bt, H), lambda i: (i, 0)),
        out_shape=jax.ShapeDtypeStruct((T, H), jnp.float32),
    )(x, yg, w, sg, su, sd)


def kernel(hidden_states, router_weight, expert_gate, expert_up, expert_down,
           shared_gate, shared_up, shared_down):
    bsz, seq, h = hidden_states.shape
    x = hidden_states.reshape(-1, h)

    topk_idx, topk_w = _router(x, router_weight)
    pos2d, meta2d = _metadata(topk_idx.reshape(_MR, _MC))
    meta = meta2d.reshape(2 * NBLK)
    pos_kt = pos2d.reshape(T, K).T
    pe = pos_kt[0].reshape(NW, DCH, CHUNK)
    po = pos_kt[1].reshape(NW, DCH, CHUNK)
    posk = pos_kt.reshape(NW, CCH, CHUNK)

    xs = _sc_dispatch(x, pe, po)
    ys = _grouped_mlp(xs, expert_gate, expert_up, expert_down, meta)
    yg = _sc_combine(ys, posk)

    out = _final(x, yg.reshape(K, T, H), topk_w,
                 shared_gate, shared_up, shared_down)
    return out.reshape(bsz, seq, h)
